# baseline (speedup 1.0000x reference)
import functools

import jax
import jax.numpy as jnp
from jax import lax
from jax.experimental import pallas as pl
from jax.experimental.pallas import tpu as pltpu
from jax.experimental.pallas import tpu_sc as plsc

NUMEL = 4096 * 4096
K = 524288
NC = 2
NS = 16
NW = NC * NS
C = 32768
NCH = NUMEL // C
CPT = NCH // NW
B = 1024
L = 16
NB = NCH + 8


def _scatter_block(chunk_ref, idx_ref, val_ref, gb, delta):
  for j in range(B // L):
    iv = idx_ref[pl.ds(j * L, L)]
    vv = val_ref[pl.ds(j * L, L)]
    loc = iv - gb
    pos = lax.iota(jnp.int32, L) + (j * L)
    inb = (loc >= 0) & (loc < C) & (pos >= delta)
    lc = jnp.minimum(jnp.maximum(loc, 0), C - 1)
    r = lc >> 12
    col = lc & 4095
    vz = jnp.where(inb, vv, 0.0)
    plsc.addupdate_scatter(chunk_ref, [r, col], vz)


def _body(t2d_hbm, idx_hbm, val_hbm, st_hbm, o2d_hbm,
          st_v, cv0, cv1, ix0, ix1, vl0, vl1,
          isem0, isem1, osem0, osem1, xsem0, xsem1, wsem0, wsem1):
  flat_hbm = t2d_hbm.reshape(NCH, 8, 4096)
  out_hbm = o2d_hbm.reshape(NCH, 8, 4096)
  cid = lax.axis_index("c")
  sid = lax.axis_index("s")
  wid = sid * NC + cid
  cbase = wid * CPT

  bufs = (cv0, cv1)
  ixs = (ix0, ix1)
  vls = (vl0, vl1)
  isems = (isem0, isem1)
  osems = (osem0, osem1)
  xsems = (xsem0, xsem1)
  wsems = (wsem0, wsem1)

  pltpu.sync_copy(st_hbm.at[pl.ds(wid * CPT, 24)], st_v.at[pl.ds(0, 24)])

  def bound_of(c):
    return st_v[pl.ds(c, L)][0]

  def gb_of(c):
    return pl.multiple_of((cbase + c) * C, C)

  def win_of(c):
    s8 = bound_of(c) & -8
    off = jnp.minimum(s8, K - B)
    return pl.multiple_of(off, 8), s8 - off

  def start_in(c, p):
    pltpu.async_copy(flat_hbm.at[cbase + c], bufs[p], isems[p])
    off, _ = win_of(c)
    pltpu.async_copy(idx_hbm.at[pl.ds(off, B)], ixs[p], xsems[p])
    pltpu.async_copy(val_hbm.at[pl.ds(off, B)], vls[p], wsems[p])

  def wait_in(p):
    pltpu.make_async_copy(flat_hbm.at[0], bufs[p], isems[p]).wait()
    pltpu.make_async_copy(idx_hbm.at[pl.ds(0, B)], ixs[p], xsems[p]).wait()
    pltpu.make_async_copy(val_hbm.at[pl.ds(0, B)], vls[p], wsems[p]).wait()

  def wait_out(p):
    pltpu.make_async_copy(bufs[p], out_hbm.at[0], osems[p]).wait()

  start_in(0, 0)

  def pair_body(g, _):
    for p in (0, 1):
      c = g * 2 + p
      q = 1 - p
      @pl.when(c >= 1)
      def _():
        wait_out(q)

      @pl.when(c + 1 < CPT)
      def _():
        start_in(c + 1, q)

      wait_in(p)

      gb = gb_of(c)
      off0, delta0 = win_of(c)
      end = bound_of(c + 1)
      nb = (end - off0 + (B - 1)) // B

      @pl.when(nb >= 1)
      def _():
        _scatter_block(bufs[p], ixs[p], vls[p], gb, delta0)

      def blk(b, __):
        raw = off0 + b * B
        off = pl.multiple_of(jnp.minimum(raw, K - B), 8)
        pltpu.sync_copy(idx_hbm.at[pl.ds(off, B)], ixs[p])
        pltpu.sync_copy(val_hbm.at[pl.ds(off, B)], vls[p])
        _scatter_block(bufs[p], ixs[p], vls[p], gb, raw - off)
        return 0

      lax.fori_loop(1, nb, blk, 0)
      pltpu.async_copy(bufs[p], out_hbm.at[cbase + c], osems[p])
    return 0

  lax.fori_loop(0, CPT // 2, pair_body, 0)
  wait_out(1)


_sc_call = functools.partial(
    pl.kernel,
    out_type=jax.ShapeDtypeStruct((4096, 4096), jnp.float32),
    mesh=plsc.VectorSubcoreMesh(
        core_axis_name="c", subcore_axis_name="s",
        num_cores=NC, num_subcores=NS),
    compiler_params=pltpu.CompilerParams(needs_layout_passes=False),
    scratch_types=[
        pltpu.VMEM((2 * L,), jnp.int32),
        pltpu.VMEM((8, 4096), jnp.float32),
        pltpu.VMEM((8, 4096), jnp.float32),
        pltpu.VMEM((B,), jnp.int32),
        pltpu.VMEM((B,), jnp.int32),
        pltpu.VMEM((B,), jnp.float32),
        pltpu.VMEM((B,), jnp.float32),
        pltpu.SemaphoreType.DMA,
        pltpu.SemaphoreType.DMA,
        pltpu.SemaphoreType.DMA,
        pltpu.SemaphoreType.DMA,
        pltpu.SemaphoreType.DMA,
        pltpu.SemaphoreType.DMA,
        pltpu.SemaphoreType.DMA,
        pltpu.SemaphoreType.DMA,
    ],
)(_body)


def kernel(tensor, values, indices):
  idx32 = indices.astype(jnp.int32)
  bounds = jnp.minimum(
      jnp.arange(NB, dtype=jnp.int32) * C, jnp.int32(NUMEL))
  pos = jnp.searchsorted(idx32, bounds, side="left").astype(jnp.int32)
  return _sc_call(tensor, idx32, values, pos)


# speedup vs baseline: 2.9455x; 2.9455x over previous
import functools

import jax
import jax.numpy as jnp
from jax import lax
from jax.experimental import pallas as pl
from jax.experimental.pallas import tpu as pltpu
from jax.experimental.pallas import tpu_sc as plsc

NUMEL = 4096 * 4096
K = 524288
NC = 2
NS = 16
NW = NC * NS
C = 32768
NCH = NUMEL // C
CPT = NCH // NW
B = 1024
L = 16
KS = K // NS


def _scatter_block(chunk_ref, idx_ref, val_ref, gb, delta):
  for j in range(B // L):
    iv = idx_ref[pl.ds(j * L, L)]
    vv = val_ref[pl.ds(j * L, L)]
    loc = iv - gb
    pos = lax.iota(jnp.int32, L) + (j * L)
    inb = (loc >= 0) & (loc < C) & (pos >= delta)
    lc = jnp.minimum(jnp.maximum(loc, 0), C - 1)
    r = lc >> 12
    col = lc & 4095
    vz = jnp.where(inb, vv, 0.0)
    plsc.addupdate_scatter(chunk_ref, [r, col], vz)


def _body(t2d_hbm, idx_hbm, val_hbm, o2d_hbm,
          pos_v, cnt_v, loc_v, stage_v, shared_cnt,
          cv0, cv1, ix0, ix1, vl0, vl1,
          isem0, isem1, osem0, osem1, xsem0, xsem1, wsem0, wsem1):
  flat_hbm = t2d_hbm.reshape(NCH, 8, 4096)
  out_hbm = o2d_hbm.reshape(NCH, 8, 4096)
  cid = lax.axis_index("c")
  sid = lax.axis_index("s")
  wid = sid * NC + cid
  cbase = wid * CPT
  lane = lax.iota(jnp.int32, L)

  pltpu.sync_copy(
      idx_hbm.at[pl.ds(pl.multiple_of(sid * KS, 8), KS)], cnt_v)

  def bs_body(g, _):
    bvec = (g * L + lane) * C
    lo = jnp.zeros((L,), jnp.int32)
    hi = jnp.full((L,), KS, jnp.int32)
    for _ in range(16):
      mid = jnp.minimum((lo + hi) >> 1, KS - 1)
      vals = plsc.load_gather(cnt_v, [mid])
      pred = vals < bvec
      lo = jnp.where(pred, mid + 1, lo)
      hi = jnp.where(pred, hi, mid)
    loc_v[pl.ds(g * L, L)] = lo
    return 0

  lax.fori_loop(0, NCH // L, bs_body, 0)

  pltpu.sync_copy(loc_v, shared_cnt.at[sid])
  plsc.subcore_barrier()
  pltpu.sync_copy(shared_cnt, stage_v)

  def sum_body(g, _):
    acc = jnp.zeros((L,), jnp.int32)
    for t in range(NS):
      acc = acc + stage_v[t, pl.ds(g * L, L)]
    pos_v[pl.ds(g * L, L)] = acc
    return 0

  lax.fori_loop(0, NCH // L, sum_body, 0)
  pos_v[pl.ds(NCH, L)] = jnp.full((L,), K, jnp.int32)

  bufs = (cv0, cv1)
  ixs = (ix0, ix1)
  vls = (vl0, vl1)
  isems = (isem0, isem1)
  osems = (osem0, osem1)
  xsems = (xsem0, xsem1)
  wsems = (wsem0, wsem1)

  def bound_of(c):
    return pos_v[pl.ds(cbase + c, L)][0]

  def gb_of(c):
    return pl.multiple_of((cbase + c) * C, C)

  def win_of(c):
    s8 = bound_of(c) & -8
    off = jnp.minimum(s8, K - B)
    return pl.multiple_of(off, 8), s8 - off

  def start_in(c, p):
    pltpu.async_copy(flat_hbm.at[cbase + c], bufs[p], isems[p])
    off, _ = win_of(c)
    pltpu.async_copy(idx_hbm.at[pl.ds(off, B)], ixs[p], xsems[p])
    pltpu.async_copy(val_hbm.at[pl.ds(off, B)], vls[p], wsems[p])

  def wait_in(p):
    pltpu.make_async_copy(flat_hbm.at[0], bufs[p], isems[p]).wait()
    pltpu.make_async_copy(idx_hbm.at[pl.ds(0, B)], ixs[p], xsems[p]).wait()
    pltpu.make_async_copy(val_hbm.at[pl.ds(0, B)], vls[p], wsems[p]).wait()

  def wait_out(p):
    pltpu.make_async_copy(bufs[p], out_hbm.at[0], osems[p]).wait()

  start_in(0, 0)

  def pair_body(g, _):
    for p in (0, 1):
      c = g * 2 + p
      q = 1 - p
      @pl.when(c >= 1)
      def _():
        wait_out(q)

      @pl.when(c + 1 < CPT)
      def _():
        start_in(c + 1, q)

      wait_in(p)

      gb = gb_of(c)
      off0, delta0 = win_of(c)
      end = bound_of(c + 1)
      nb = (end - off0 + (B - 1)) // B

      @pl.when(nb >= 1)
      def _():
        _scatter_block(bufs[p], ixs[p], vls[p], gb, delta0)

      def blk(b, __):
        raw = off0 + b * B
        off = pl.multiple_of(jnp.minimum(raw, K - B), 8)
        pltpu.sync_copy(idx_hbm.at[pl.ds(off, B)], ixs[p])
        pltpu.sync_copy(val_hbm.at[pl.ds(off, B)], vls[p])
        _scatter_block(bufs[p], ixs[p], vls[p], gb, raw - off)
        return 0

      lax.fori_loop(1, nb, blk, 0)
      pltpu.async_copy(bufs[p], out_hbm.at[cbase + c], osems[p])
    return 0

  lax.fori_loop(0, CPT // 2, pair_body, 0)
  wait_out(1)


_sc_call = functools.partial(
    pl.kernel,
    out_type=jax.ShapeDtypeStruct((4096, 4096), jnp.float32),
    mesh=plsc.VectorSubcoreMesh(
        core_axis_name="c", subcore_axis_name="s",
        num_cores=NC, num_subcores=NS),
    compiler_params=pltpu.CompilerParams(needs_layout_passes=False),
    scratch_types=[
        pltpu.VMEM((NCH + L,), jnp.int32),
        pltpu.VMEM((KS,), jnp.int32),
        pltpu.VMEM((NCH,), jnp.int32),
        pltpu.VMEM((NS, NCH), jnp.int32),
        pltpu.VMEM_SHARED((NS, NCH), jnp.int32),
        pltpu.VMEM((8, 4096), jnp.float32),
        pltpu.VMEM((8, 4096), jnp.float32),
        pltpu.VMEM((B,), jnp.int32),
        pltpu.VMEM((B,), jnp.int32),
        pltpu.VMEM((B,), jnp.float32),
        pltpu.VMEM((B,), jnp.float32),
        pltpu.SemaphoreType.DMA,
        pltpu.SemaphoreType.DMA,
        pltpu.SemaphoreType.DMA,
        pltpu.SemaphoreType.DMA,
        pltpu.SemaphoreType.DMA,
        pltpu.SemaphoreType.DMA,
        pltpu.SemaphoreType.DMA,
        pltpu.SemaphoreType.DMA,
    ],
)(_body)


def kernel(tensor, values, indices):
  idx32 = indices.astype(jnp.int32)
  return _sc_call(tensor, idx32, values)


# speedup vs baseline: 3.2618x; 1.1074x over previous
import functools

import jax
import jax.numpy as jnp
from jax import lax
from jax.experimental import pallas as pl
from jax.experimental.pallas import tpu as pltpu
from jax.experimental.pallas import tpu_sc as plsc

NUMEL = 4096 * 4096
K = 524288
NC = 2
NS = 16
NW = NC * NS
C = 32768
NCH = NUMEL // C
CPT = NCH // NW
B = 2048
L = 16
KS = K // NS


def _scatter_block(chunk_ref, idx_ref, val_ref, gb, delta):
  for j in range(B // L):
    iv = idx_ref[pl.ds(j * L, L)]
    vv = val_ref[pl.ds(j * L, L)]
    loc = iv - gb
    pos = lax.iota(jnp.int32, L) + (j * L)
    inb = (loc >= 0) & (loc < C) & (pos >= delta)
    lc = jnp.minimum(jnp.maximum(loc, 0), C - 1)
    r = lc >> 12
    col = lc & 4095
    vz = jnp.where(inb, vv, 0.0)
    plsc.addupdate_scatter(chunk_ref, [r, col], vz)


def _body(t2d_hbm, idx_hbm, val_hbm, o2d_hbm,
          pos_v, cnt_v, loc_v, stage_v, shared_cnt,
          cv0, cv1, ix0, ix1, vl0, vl1,
          isem0, isem1, osem0, osem1, xsem0, xsem1, wsem0, wsem1):
  flat_hbm = t2d_hbm.reshape(NCH, 8, 4096)
  out_hbm = o2d_hbm.reshape(NCH, 8, 4096)
  cid = lax.axis_index("c")
  sid = lax.axis_index("s")
  wid = sid * NC + cid
  cbase = wid * CPT
  lane = lax.iota(jnp.int32, L)

  pltpu.async_copy(flat_hbm.at[cbase], cv0, isem0)
  pltpu.async_copy(flat_hbm.at[cbase + 1], cv1, isem1)

  pltpu.sync_copy(
      idx_hbm.at[pl.ds(pl.multiple_of(sid * KS, 8), KS)], cnt_v)

  def bs_body(g, _):
    bvec = (g * L + lane) * C
    lo = jnp.zeros((L,), jnp.int32)
    hi = jnp.full((L,), KS, jnp.int32)
    for _ in range(16):
      mid = jnp.minimum((lo + hi) >> 1, KS - 1)
      vals = plsc.load_gather(cnt_v, [mid])
      pred = vals < bvec
      lo = jnp.where(pred, mid + 1, lo)
      hi = jnp.where(pred, hi, mid)
    loc_v[pl.ds(g * L, L)] = lo
    return 0

  lax.fori_loop(0, NCH // L, bs_body, 0)

  pltpu.sync_copy(loc_v, shared_cnt.at[sid])
  plsc.subcore_barrier()
  pltpu.sync_copy(shared_cnt, stage_v)

  def sum_body(g, _):
    acc = jnp.zeros((L,), jnp.int32)
    for t in range(NS):
      acc = acc + stage_v[t, pl.ds(g * L, L)]
    pos_v[pl.ds(g * L, L)] = acc
    return 0

  lax.fori_loop(0, NCH // L, sum_body, 0)
  pos_v[pl.ds(NCH, L)] = jnp.full((L,), K, jnp.int32)

  bufs = (cv0, cv1)
  ixs = (ix0, ix1)
  vls = (vl0, vl1)
  isems = (isem0, isem1)
  osems = (osem0, osem1)
  xsems = (xsem0, xsem1)
  wsems = (wsem0, wsem1)

  def bound_of(c):
    return pos_v[pl.ds(cbase + c, L)][0]

  def gb_of(c):
    return pl.multiple_of((cbase + c) * C, C)

  def win_of(c):
    s8 = bound_of(c) & -8
    off = jnp.minimum(s8, K - B)
    return pl.multiple_of(off, 8), s8 - off

  def start_dense(c, p):
    pltpu.async_copy(flat_hbm.at[cbase + c], bufs[p], isems[p])

  def start_idx(c, p):
    off, _ = win_of(c)
    pltpu.async_copy(idx_hbm.at[pl.ds(off, B)], ixs[p], xsems[p])
    pltpu.async_copy(val_hbm.at[pl.ds(off, B)], vls[p], wsems[p])

  def wait_in(p):
    pltpu.make_async_copy(flat_hbm.at[0], bufs[p], isems[p]).wait()
    pltpu.make_async_copy(idx_hbm.at[pl.ds(0, B)], ixs[p], xsems[p]).wait()
    pltpu.make_async_copy(val_hbm.at[pl.ds(0, B)], vls[p], wsems[p]).wait()

  def wait_out(p):
    pltpu.make_async_copy(bufs[p], out_hbm.at[0], osems[p]).wait()

  start_idx(0, 0)

  def pair_body(g, _):
    for p in (0, 1):
      c = g * 2 + p
      q = 1 - p
      @pl.when(c >= 1)
      def _():
        wait_out(q)

      @pl.when(c + 1 < CPT)
      def _():
        @pl.when(c >= 1)
        def _():
          start_dense(c + 1, q)

        start_idx(c + 1, q)

      wait_in(p)

      gb = gb_of(c)
      off0, delta0 = win_of(c)
      end = bound_of(c + 1)
      nb = (end - off0 + (B - 1)) // B

      @pl.when(nb >= 1)
      def _():
        _scatter_block(bufs[p], ixs[p], vls[p], gb, delta0)

      def blk(b, __):
        raw = off0 + b * B
        off = pl.multiple_of(jnp.minimum(raw, K - B), 8)
        pltpu.sync_copy(idx_hbm.at[pl.ds(off, B)], ixs[p])
        pltpu.sync_copy(val_hbm.at[pl.ds(off, B)], vls[p])
        _scatter_block(bufs[p], ixs[p], vls[p], gb, raw - off)
        return 0

      lax.fori_loop(1, nb, blk, 0)
      pltpu.async_copy(bufs[p], out_hbm.at[cbase + c], osems[p])
    return 0

  lax.fori_loop(0, CPT // 2, pair_body, 0)
  wait_out(1)


_sc_call = functools.partial(
    pl.kernel,
    out_type=jax.ShapeDtypeStruct((4096, 4096), jnp.float32),
    mesh=plsc.VectorSubcoreMesh(
        core_axis_name="c", subcore_axis_name="s",
        num_cores=NC, num_subcores=NS),
    compiler_params=pltpu.CompilerParams(needs_layout_passes=False),
    scratch_types=[
        pltpu.VMEM((NCH + L,), jnp.int32),
        pltpu.VMEM((KS,), jnp.int32),
        pltpu.VMEM((NCH,), jnp.int32),
        pltpu.VMEM((NS, NCH), jnp.int32),
        pltpu.VMEM_SHARED((NS, NCH), jnp.int32),
        pltpu.VMEM((8, 4096), jnp.float32),
        pltpu.VMEM((8, 4096), jnp.float32),
        pltpu.VMEM((B,), jnp.int32),
        pltpu.VMEM((B,), jnp.int32),
        pltpu.VMEM((B,), jnp.float32),
        pltpu.VMEM((B,), jnp.float32),
        pltpu.SemaphoreType.DMA,
        pltpu.SemaphoreType.DMA,
        pltpu.SemaphoreType.DMA,
        pltpu.SemaphoreType.DMA,
        pltpu.SemaphoreType.DMA,
        pltpu.SemaphoreType.DMA,
        pltpu.SemaphoreType.DMA,
        pltpu.SemaphoreType.DMA,
    ],
)(_body)


def kernel(tensor, values, indices):
  idx32 = indices.astype(jnp.int32)
  return _sc_call(tensor, idx32, values)


# speedup vs baseline: 4.9678x; 1.5230x over previous
import functools

import jax
import jax.numpy as jnp
from jax import lax
from jax.experimental import pallas as pl
from jax.experimental.pallas import tpu as pltpu
from jax.experimental.pallas import tpu_sc as plsc

NUMEL = 4096 * 4096
K = 524288
NC = 2
NS = 16
NW = NC * NS
C = 32768
NCH = NUMEL // C
CPT = NCH // NW
B = 2048
L = 16
KS = K // NS


def _scatter_block(chunk_ref, idx_ref, val_ref, gb, delta, ng):
  def group(j, _):
    iv = idx_ref[pl.ds(j * L, L)]
    vv = val_ref[pl.ds(j * L, L)]
    loc = iv - gb
    pos = lax.iota(jnp.int32, L) + (j * L)
    inb = (loc >= 0) & (loc < C) & (pos >= delta)
    lc = jnp.minimum(jnp.maximum(loc, 0), C - 1)
    r = lc >> 12
    col = lc & 4095
    vz = jnp.where(inb, vv, 0.0)
    plsc.addupdate_scatter(chunk_ref, [r, col], vz)
    return 0

  lax.fori_loop(0, ng, group, 0)


def _body(t2d_hbm, idx_hbm, val_hbm, o2d_hbm,
          pos_v, cnt_v, loc_v, stage_v, shared_cnt,
          cv0, cv1, ix0, ix1, vl0, vl1,
          isem0, isem1, osem0, osem1, xsem0, xsem1, wsem0, wsem1):
  flat_hbm = t2d_hbm.reshape(NCH, 8, 4096)
  out_hbm = o2d_hbm.reshape(NCH, 8, 4096)
  cid = lax.axis_index("c")
  sid = lax.axis_index("s")
  wid = sid * NC + cid
  cbase = wid * CPT
  lane = lax.iota(jnp.int32, L)

  pltpu.async_copy(flat_hbm.at[cbase], cv0, isem0)
  pltpu.async_copy(flat_hbm.at[cbase + 1], cv1, isem1)

  pltpu.sync_copy(
      idx_hbm.at[pl.ds(pl.multiple_of(sid * KS, 8), KS)], cnt_v)

  def bs_body(g, _):
    bvec = (g * L + lane) * C
    lo = jnp.zeros((L,), jnp.int32)
    hi = jnp.full((L,), KS, jnp.int32)
    for _ in range(16):
      mid = jnp.minimum((lo + hi) >> 1, KS - 1)
      vals = plsc.load_gather(cnt_v, [mid])
      pred = vals < bvec
      lo = jnp.where(pred, mid + 1, lo)
      hi = jnp.where(pred, hi, mid)
    loc_v[pl.ds(g * L, L)] = lo
    return 0

  lax.fori_loop(0, NCH // L, bs_body, 0)

  pltpu.sync_copy(loc_v, shared_cnt.at[sid])
  plsc.subcore_barrier()
  pltpu.sync_copy(shared_cnt, stage_v)

  def sum_body(g, _):
    acc = jnp.zeros((L,), jnp.int32)
    for t in range(NS):
      acc = acc + stage_v[t, pl.ds(g * L, L)]
    pos_v[pl.ds(g * L, L)] = acc
    return 0

  lax.fori_loop(0, NCH // L, sum_body, 0)
  pos_v[pl.ds(NCH, L)] = jnp.full((L,), K, jnp.int32)

  bufs = (cv0, cv1)
  ixs = (ix0, ix1)
  vls = (vl0, vl1)
  isems = (isem0, isem1)
  osems = (osem0, osem1)
  xsems = (xsem0, xsem1)
  wsems = (wsem0, wsem1)

  def bound_of(c):
    return pos_v[pl.ds(cbase + c, L)][0]

  def gb_of(c):
    return pl.multiple_of((cbase + c) * C, C)

  def win_of(c):
    s8 = bound_of(c) & -8
    off = jnp.minimum(s8, K - B)
    return pl.multiple_of(off, 8), s8 - off

  def start_dense(c, p):
    pltpu.async_copy(flat_hbm.at[cbase + c], bufs[p], isems[p])

  def start_idx(c, p):
    off, _ = win_of(c)
    pltpu.async_copy(idx_hbm.at[pl.ds(off, B)], ixs[p], xsems[p])
    pltpu.async_copy(val_hbm.at[pl.ds(off, B)], vls[p], wsems[p])

  def wait_in(p):
    pltpu.make_async_copy(flat_hbm.at[0], bufs[p], isems[p]).wait()
    pltpu.make_async_copy(idx_hbm.at[pl.ds(0, B)], ixs[p], xsems[p]).wait()
    pltpu.make_async_copy(val_hbm.at[pl.ds(0, B)], vls[p], wsems[p]).wait()

  def wait_out(p):
    pltpu.make_async_copy(bufs[p], out_hbm.at[0], osems[p]).wait()

  start_idx(0, 0)

  def pair_body(g, _):
    for p in (0, 1):
      c = g * 2 + p
      q = 1 - p
      @pl.when(c >= 1)
      def _():
        wait_out(q)

      @pl.when(c + 1 < CPT)
      def _():
        @pl.when(c >= 1)
        def _():
          start_dense(c + 1, q)

        start_idx(c + 1, q)

      wait_in(p)

      gb = gb_of(c)
      off0, delta0 = win_of(c)
      end = bound_of(c + 1)
      nb = (end - off0 + (B - 1)) // B

      def groups_from(off):
        return jnp.clip((end - off + (L - 1)) // L, 0, B // L)

      _scatter_block(bufs[p], ixs[p], vls[p], gb, delta0, groups_from(off0))

      def blk(b, __):
        raw = off0 + b * B
        off = pl.multiple_of(jnp.minimum(raw, K - B), 8)
        pltpu.sync_copy(idx_hbm.at[pl.ds(off, B)], ixs[p])
        pltpu.sync_copy(val_hbm.at[pl.ds(off, B)], vls[p])
        _scatter_block(bufs[p], ixs[p], vls[p], gb, raw - off,
                       groups_from(off))
        return 0

      lax.fori_loop(1, nb, blk, 0)
      pltpu.async_copy(bufs[p], out_hbm.at[cbase + c], osems[p])
    return 0

  lax.fori_loop(0, CPT // 2, pair_body, 0)
  wait_out(1)


_sc_call = functools.partial(
    pl.kernel,
    out_type=jax.ShapeDtypeStruct((4096, 4096), jnp.float32),
    mesh=plsc.VectorSubcoreMesh(
        core_axis_name="c", subcore_axis_name="s",
        num_cores=NC, num_subcores=NS),
    compiler_params=pltpu.CompilerParams(needs_layout_passes=False),
    scratch_types=[
        pltpu.VMEM((NCH + L,), jnp.int32),
        pltpu.VMEM((KS,), jnp.int32),
        pltpu.VMEM((NCH,), jnp.int32),
        pltpu.VMEM((NS, NCH), jnp.int32),
        pltpu.VMEM_SHARED((NS, NCH), jnp.int32),
        pltpu.VMEM((8, 4096), jnp.float32),
        pltpu.VMEM((8, 4096), jnp.float32),
        pltpu.VMEM((B,), jnp.int32),
        pltpu.VMEM((B,), jnp.int32),
        pltpu.VMEM((B,), jnp.float32),
        pltpu.VMEM((B,), jnp.float32),
        pltpu.SemaphoreType.DMA,
        pltpu.SemaphoreType.DMA,
        pltpu.SemaphoreType.DMA,
        pltpu.SemaphoreType.DMA,
        pltpu.SemaphoreType.DMA,
        pltpu.SemaphoreType.DMA,
        pltpu.SemaphoreType.DMA,
        pltpu.SemaphoreType.DMA,
    ],
)(_body)


def kernel(tensor, values, indices):
  idx32 = indices.astype(jnp.int32)
  return _sc_call(tensor, idx32, values)
